# Initial kernel scaffold; baseline (speedup 1.0000x reference)
import jax
import jax.numpy as jnp
from jax.experimental import pallas as pl


def kernel(x, table, pos_vectors):
    raise NotImplementedError("write your pallas kernel here")


# speedup vs baseline: 3.4633x; 3.4633x over previous
import functools

import jax
import jax.numpy as jnp
from jax import lax
from jax.experimental import pallas as pl
from jax.experimental.pallas import tpu as pltpu
from jax.experimental.pallas import tpu_sc as plsc

_NC = 2
_NS = 16
_NW = _NC * _NS
_CH = 128


@functools.lru_cache(maxsize=None)
def _make_gather(N: int, V: int, D: int):
    per_w = N // _NW
    n_ch = per_w // _CH
    mesh = plsc.VectorSubcoreMesh(core_axis_name="c", subcore_axis_name="s")

    @functools.partial(
        pl.kernel,
        out_type=jax.ShapeDtypeStruct((N, D), jnp.float32),
        mesh=mesh,
        scratch_types=[
            pltpu.VMEM((_CH,), jnp.int32),
            pltpu.VMEM((_CH, D), jnp.float32),
            pltpu.SemaphoreType.DMA,
        ],
        compiler_params=pltpu.CompilerParams(use_tc_tiling_on_sc=False),
    )
    def gather(tab_hbm, idx_hbm, out_hbm, idx_v, rows_v, sem):
        wid = lax.axis_index("s") * _NC + lax.axis_index("c")
        base0 = wid * per_w

        def body(c, _):
            base = base0 + c * _CH
            pltpu.sync_copy(idx_hbm.at[pl.ds(base, _CH)], idx_v)
            pltpu.async_copy(tab_hbm.at[idx_v], rows_v, sem).wait()
            pltpu.sync_copy(rows_v, out_hbm.at[pl.ds(base, _CH)])
            return 0

        lax.fori_loop(0, n_ch, body, 0)

    return gather


def kernel(x, table, pos_vectors):
    B, S = x.shape
    V, H = table.shape
    P = pos_vectors.shape[0]
    D = P + H
    comb = jnp.concatenate([pos_vectors.T, table], axis=1)
    idx = x.reshape(-1).astype(jnp.int32)
    out = _make_gather(B * S, V, D)(comb, idx)
    return out.reshape(B, S, D)


# speedup vs baseline: 4.3425x; 1.2539x over previous
import functools

import jax
import jax.numpy as jnp
from jax import lax
from jax.experimental import pallas as pl
from jax.experimental.pallas import tpu as pltpu
from jax.experimental.pallas import tpu_sc as plsc

_NC = 2
_NS = 16
_NW = _NC * _NS
_CH = 128
_NBUF = 5


@functools.lru_cache(maxsize=None)
def _make_gather(N: int, V: int, H: int, P: int):
    D = P + H
    per_w = N // _NW
    n_ch = per_w // _CH
    n_outer = n_ch // _NBUF
    assert per_w % _CH == 0 and n_ch % _NBUF == 0
    mesh = plsc.VectorSubcoreMesh(core_axis_name="c", subcore_axis_name="s")

    @functools.partial(
        pl.kernel,
        out_type=jax.ShapeDtypeStruct((N, D), jnp.float32),
        mesh=mesh,
        scratch_types=(
            [pltpu.VMEM((n_ch, _CH), jnp.int32)]
            + [pltpu.VMEM((_CH, H), jnp.float32) for _ in range(_NBUF)]
            + [pltpu.VMEM((_CH, P), jnp.float32) for _ in range(_NBUF)]
            + [pltpu.SemaphoreType.DMA for _ in range(4 * _NBUF)]
        ),
        compiler_params=pltpu.CompilerParams(use_tc_tiling_on_sc=False),
    )
    def gather(tab_hbm, pos_hbm, idx_hbm, out_hbm, idx_v, *bufs):
        trows = bufs[:_NBUF]
        prows = bufs[_NBUF:2 * _NBUF]
        gsem_t = bufs[2 * _NBUF:3 * _NBUF]
        gsem_p = bufs[3 * _NBUF:4 * _NBUF]
        wsem_t = bufs[4 * _NBUF:5 * _NBUF]
        wsem_p = bufs[5 * _NBUF:6 * _NBUF]

        wid = lax.axis_index("s") * _NC + lax.axis_index("c")
        base0 = wid * per_w
        row0 = wid * n_ch

        pltpu.sync_copy(idx_hbm.at[pl.ds(row0, n_ch)], idx_v)

        def start_gathers(c, b):
            pltpu.async_copy(tab_hbm.at[idx_v.at[c]], trows[b], gsem_t[b])
            pltpu.async_copy(pos_hbm.at[idx_v.at[c]], prows[b], gsem_p[b])

        for b in range(_NBUF):
            start_gathers(b, b)

        def outer(g, carry):
            c0 = g * _NBUF
            for b in range(_NBUF):
                c = c0 + b
                base = base0 + c * _CH
                pltpu.make_async_copy(
                    tab_hbm.at[idx_v.at[c]], trows[b], gsem_t[b]).wait()
                pltpu.make_async_copy(
                    pos_hbm.at[idx_v.at[c]], prows[b], gsem_p[b]).wait()
                pltpu.async_copy(
                    trows[b], out_hbm.at[pl.ds(base, _CH), pl.ds(P, H)],
                    wsem_t[b])
                pltpu.async_copy(
                    prows[b], out_hbm.at[pl.ds(base, _CH), pl.ds(0, P)],
                    wsem_p[b])
            for b in range(_NBUF):
                c = c0 + b
                base = base0 + c * _CH
                pltpu.make_async_copy(
                    trows[b], out_hbm.at[pl.ds(base, _CH), pl.ds(P, H)],
                    wsem_t[b]).wait()
                pltpu.make_async_copy(
                    prows[b], out_hbm.at[pl.ds(base, _CH), pl.ds(0, P)],
                    wsem_p[b]).wait()

                @pl.when(g < n_outer - 1)
                def _():
                    start_gathers(c + _NBUF, b)

            return carry

        lax.fori_loop(0, n_outer, outer, 0)

    return gather


def kernel(x, table, pos_vectors):
    B, S = x.shape
    V, H = table.shape
    P = pos_vectors.shape[0]
    N = B * S
    pos_t = pos_vectors.T
    idx = x.reshape(N // _CH, _CH).astype(jnp.int32)
    out = _make_gather(N, V, H, P)(table, pos_t, idx)
    return out.reshape(B, S, P + H)
